# baseline (speedup 1.0000x reference)
import functools

import jax
import jax.numpy as jnp
from jax import lax
from jax.experimental import pallas as pl
from jax.experimental.pallas import tpu as pltpu
from jax.experimental.pallas import tpu_sc as plsc

NN = 10000
NE = 320000
MD = 128
AD = 128
BD = 16

NC = 2
NS = 16
NW = NC * NS
EPW = NE // NW
CHUNK = 80
NITER = EPW // CHUNK


def _sc_mesh():
    return plsc.VectorSubcoreMesh(core_axis_name="c", subcore_axis_name="s")


_NBUF = 3


def _scatter_add(msg, dst3, zeros):
    @functools.partial(
        pl.kernel,
        mesh=_sc_mesh(),
        out_type=jax.ShapeDtypeStruct((NC * NN, MD), jnp.float32),
        scratch_types=[
            pltpu.VMEM((NITER, CHUNK), jnp.int32),
            *[pltpu.VMEM((CHUNK, MD), jnp.float32) for _ in range(_NBUF)],
            *[pltpu.SemaphoreType.DMA for _ in range(_NBUF)],
            pltpu.VMEM_SHARED((NN, MD), jnp.float32),
        ],
    )
    def k(msg_hbm, dst_hbm, zeros_hbm, out_hbm, idx_v, *rest):
        bufs = rest[:_NBUF]
        sems = rest[_NBUF:2 * _NBUF]
        inc_sh = rest[2 * _NBUF]
        cid = lax.axis_index("c")
        sid = lax.axis_index("s")
        wid = sid * NC + cid
        rpt = 1000

        @pl.when(sid < NN // rpt)
        def _():
            pltpu.sync_copy(zeros_hbm.at[pl.ds(sid * rpt, rpt)],
                            inc_sh.at[pl.ds(sid * rpt, rpt)])

        pltpu.sync_copy(dst_hbm.at[wid], idx_v)
        plsc.subcore_barrier()

        loads = [None] * _NBUF

        def start_load(b, i):
            base = wid * EPW + i * CHUNK
            loads[b] = pltpu.async_copy(
                msg_hbm.at[pl.ds(base, CHUNK)], bufs[b], sems[b])

        for b in range(_NBUF):
            start_load(b, b)
        for i in range(NITER):
            b = i % _NBUF
            loads[b].wait()
            pltpu.sync_copy(bufs[b], inc_sh.at[idx_v.at[i]], add=True)
            if i + _NBUF < NITER:
                start_load(b, i + _NBUF)

        plsc.subcore_barrier()

        @pl.when(sid < NN // rpt)
        def _():
            pltpu.sync_copy(inc_sh.at[pl.ds(sid * rpt, rpt)],
                            out_hbm.at[pl.ds(cid * NN + sid * rpt, rpt)])

    return k(msg, dst3, zeros)


_GRP = 5
_NG = NITER // _GRP
_GROWS = _GRP * CHUNK


def _gather_rows(nf, src3):
    @functools.partial(
        pl.kernel,
        mesh=_sc_mesh(),
        compiler_params=pltpu.CompilerParams(use_tc_tiling_on_sc=False),
        out_type=jax.ShapeDtypeStruct((NE, MD // 2), jnp.int32),
        scratch_types=[
            pltpu.VMEM((NITER, CHUNK), jnp.int32),
            pltpu.VMEM((_GROWS, MD // 2), jnp.int32),
            pltpu.VMEM((_GROWS, MD // 2), jnp.int32),
            pltpu.SemaphoreType.DMA,
            pltpu.SemaphoreType.DMA,
            pltpu.SemaphoreType.DMA,
        ],
    )
    def k(nf_hbm, src_hbm, out_hbm, idx_v, rows0, rows1, gsem, w0, w1):
        cid = lax.axis_index("c")
        sid = lax.axis_index("s")
        wid = sid * NC + cid
        pltpu.sync_copy(src_hbm.at[wid], idx_v)

        rows = (rows0, rows1)
        wsem = (w0, w1)
        wh = [None, None]
        for g in range(_NG):
            r = g & 1
            if wh[r] is not None:
                wh[r].wait()
            ghs = []
            for b in range(_GRP):
                i = g * _GRP + b
                ghs.append(pltpu.async_copy(
                    nf_hbm.at[idx_v.at[i]],
                    rows[r].at[pl.ds(b * CHUNK, CHUNK)], gsem))
            for h in ghs:
                h.wait()
            base = wid * EPW + g * _GROWS
            wh[r] = pltpu.async_copy(
                rows[r], out_hbm.at[pl.ds(base, _GROWS)], wsem[r])
        for h in wh:
            h.wait()

    return k(nf, src3)


_NF_BLK = 2000
_HD = MD // 2


def _rne_bf16_bits(f):
    u = lax.bitcast_convert_type(f, jnp.uint32)
    return ((u + jnp.uint32(0x7FFF) + ((u >> 16) & jnp.uint32(1))) >> 16
            ).astype(jnp.int32)


def _node_features(inc0, inc1, atom, WaT, WcT, b):
    def body(i0_ref, i1_ref, a_ref, wa_ref, wc_ref, b_ref, o_ref):
        inc = i0_ref[...] + i1_ref[...]
        nf = (
            jnp.dot(a_ref[...], wa_ref[...], preferred_element_type=jnp.float32)
            + jnp.dot(inc, wc_ref[...], preferred_element_type=jnp.float32)
            + b_ref[...]
        )
        lo = _rne_bf16_bits(nf[:, :_HD])
        hi = _rne_bf16_bits(nf[:, _HD:])
        o_ref[...] = (hi << 16) | lo

    blk = pl.BlockSpec((_NF_BLK, MD), lambda i: (i, 0))
    full = pl.BlockSpec((MD, MD), lambda i: (0, 0))
    vec = pl.BlockSpec((1, MD), lambda i: (0, 0))
    return pl.pallas_call(
        body,
        grid=(NN // _NF_BLK,),
        in_specs=[blk, blk, blk, full, full, vec],
        out_specs=pl.BlockSpec((_NF_BLK, _HD), lambda i: (i, 0)),
        out_shape=jax.ShapeDtypeStruct((NN, _HD), jnp.int32),
    )(inc0, inc1, atom, WaT, WcT, b)


_ED_BLK = 2000


def _edge_epilogue(msg, gathered, bond, WbT, gamma, beta):
    def body(m_ref, g_ref, bo_ref, wb_ref, ga_ref, be_ref, o_ref):
        bp = jnp.dot(bo_ref[...], wb_ref[...], preferred_element_type=jnp.float32)
        w = g_ref[...]
        f_lo = lax.bitcast_convert_type(w << 16, jnp.float32)
        f_hi = lax.bitcast_convert_type(w & jnp.int32(-65536), jnp.float32)
        nf_rows = jnp.concatenate([f_lo, f_hi], axis=1)
        upd = jnp.maximum(nf_rows + bp, 0.0)
        x = m_ref[...] + upd
        mu = jnp.mean(x, axis=1, keepdims=True)
        xc = x - mu
        var = jnp.mean(xc * xc, axis=1, keepdims=True)
        o_ref[...] = xc * lax.rsqrt(var + 1e-5) * ga_ref[...] + be_ref[...]

    blk = pl.BlockSpec((_ED_BLK, MD), lambda i: (i, 0))
    gblk = pl.BlockSpec((_ED_BLK, _HD), lambda i: (i, 0))
    bblk = pl.BlockSpec((_ED_BLK, BD), lambda i: (i, 0))
    wblk = pl.BlockSpec((BD, MD), lambda i: (0, 0))
    vec = pl.BlockSpec((1, MD), lambda i: (0, 0))
    return pl.pallas_call(
        body,
        grid=(NE // _ED_BLK,),
        in_specs=[blk, gblk, bblk, wblk, vec, vec],
        out_specs=blk,
        out_shape=jax.ShapeDtypeStruct((NE, MD), jnp.float32),
    )(msg, gathered, bond, WbT, gamma, beta)


def kernel(msg, atom, bond, src, dst, W, b, gamma, beta):
    src3 = src.astype(jnp.int32).reshape(NW, NITER, CHUNK)
    dst3 = dst.astype(jnp.int32).reshape(NW, NITER, CHUNK)
    WaT = W[:, :AD].T
    WbT = W[:, AD:AD + BD].T
    WcT = W[:, AD + BD:].T
    zeros = jnp.zeros((NN, MD), jnp.float32)

    inc2 = _scatter_add(msg, dst3, zeros)
    nf = _node_features(inc2[:NN], inc2[NN:], atom, WaT, WcT,
                        b.reshape(1, MD))
    gathered = _gather_rows(nf, src3)
    return _edge_epilogue(msg, gathered, bond, WbT,
                          gamma.reshape(1, MD), beta.reshape(1, MD))


# speedup vs baseline: 1.1383x; 1.1383x over previous
import functools

import jax
import jax.numpy as jnp
from jax import lax
from jax.experimental import pallas as pl
from jax.experimental.pallas import tpu as pltpu
from jax.experimental.pallas import tpu_sc as plsc

NN = 10000
NE = 320000
MD = 128
AD = 128
BD = 16

NC = 2
NS = 16
NW = NC * NS
EPW = NE // NW
CHUNK = 80
NITER = EPW // CHUNK


def _sc_mesh():
    return plsc.VectorSubcoreMesh(core_axis_name="c", subcore_axis_name="s")


_NBUF = 3


def _scatter_add(msg, dst3, zeros):
    @functools.partial(
        pl.kernel,
        mesh=_sc_mesh(),
        out_type=jax.ShapeDtypeStruct((NC * NN, MD), jnp.float32),
        scratch_types=[
            pltpu.VMEM((NITER, CHUNK), jnp.int32),
            *[pltpu.VMEM((CHUNK, MD), jnp.float32) for _ in range(_NBUF)],
            *[pltpu.SemaphoreType.DMA for _ in range(_NBUF)],
            pltpu.VMEM_SHARED((NN, MD), jnp.float32),
        ],
    )
    def k(msg_hbm, dst_hbm, zeros_hbm, out_hbm, idx_v, *rest):
        bufs = rest[:_NBUF]
        sems = rest[_NBUF:2 * _NBUF]
        inc_sh = rest[2 * _NBUF]
        cid = lax.axis_index("c")
        sid = lax.axis_index("s")
        wid = sid * NC + cid
        rpt = 1000

        @pl.when(sid < NN // rpt)
        def _():
            pltpu.sync_copy(zeros_hbm.at[pl.ds(sid * rpt, rpt)],
                            inc_sh.at[pl.ds(sid * rpt, rpt)])

        pltpu.sync_copy(dst_hbm.at[wid], idx_v)
        plsc.subcore_barrier()

        loads = [None] * _NBUF

        def start_load(b, i):
            base = wid * EPW + i * CHUNK
            loads[b] = pltpu.async_copy(
                msg_hbm.at[pl.ds(base, CHUNK)], bufs[b], sems[b])

        for b in range(_NBUF):
            start_load(b, b)
        for i in range(NITER):
            b = i % _NBUF
            loads[b].wait()
            pltpu.sync_copy(bufs[b], inc_sh.at[idx_v.at[i]], add=True)
            if i + _NBUF < NITER:
                start_load(b, i + _NBUF)

        plsc.subcore_barrier()

        @pl.when(sid < NN // rpt)
        def _():
            pltpu.sync_copy(inc_sh.at[pl.ds(sid * rpt, rpt)],
                            out_hbm.at[pl.ds(cid * NN + sid * rpt, rpt)])

    return k(msg, dst3, zeros)


NEH = NE // 2
EPH = NEH // NW
_CHS = [80] * 62 + [40]
_GRP = 5
_GROWS = 400


def _gather_half(nf, srcw):
    groups, off, ci = [], 0, 0
    while ci < len(_CHS):
        chs = _CHS[ci:ci + _GRP]
        chunks, coff = [], off
        for s in chs:
            chunks.append((coff, s))
            coff += s
        groups.append((off, chunks, coff - off))
        off, ci = coff, ci + len(chs)

    @functools.partial(
        pl.kernel,
        mesh=_sc_mesh(),
        out_type=jax.ShapeDtypeStruct((NEH, MD), jnp.float32),
        scratch_types=[
            pltpu.VMEM((EPH,), jnp.int32),
            pltpu.VMEM((_GROWS, MD), jnp.float32),
            pltpu.VMEM((_GROWS, MD), jnp.float32),
            pltpu.SemaphoreType.DMA,
            pltpu.SemaphoreType.DMA,
            pltpu.SemaphoreType.DMA,
        ],
    )
    def k(nf_hbm, src_hbm, out_hbm, idx_v, rows0, rows1, gsem, w0, w1):
        cid = lax.axis_index("c")
        sid = lax.axis_index("s")
        wid = sid * NC + cid
        pltpu.sync_copy(src_hbm.at[wid], idx_v)

        rows = (rows0, rows1)
        wsem = (w0, w1)
        wh = [None, None]
        for gi, (goff, chunks, grows) in enumerate(groups):
            r = gi & 1
            if wh[r] is not None:
                wh[r].wait()
            ghs, boff = [], 0
            for coff, csz in chunks:
                ghs.append(pltpu.async_copy(
                    nf_hbm.at[idx_v.at[pl.ds(coff, csz)]],
                    rows[r].at[pl.ds(boff, csz)], gsem))
                boff += csz
            for h in ghs:
                h.wait()
            wh[r] = pltpu.async_copy(
                rows[r].at[pl.ds(0, grows)],
                out_hbm.at[pl.ds(wid * EPH + goff, grows)], wsem[r])
        for h in wh:
            if h is not None:
                h.wait()

    return k(nf, srcw)


_NF_BLK = 2000


def _node_features(inc0, inc1, atom, WaT, WcT, b):
    def body(i0_ref, i1_ref, a_ref, wa_ref, wc_ref, b_ref, o_ref):
        inc = i0_ref[...] + i1_ref[...]
        o_ref[...] = (
            jnp.dot(a_ref[...], wa_ref[...], preferred_element_type=jnp.float32)
            + jnp.dot(inc, wc_ref[...], preferred_element_type=jnp.float32)
            + b_ref[...]
        )

    blk = pl.BlockSpec((_NF_BLK, MD), lambda i: (i, 0))
    full = pl.BlockSpec((MD, MD), lambda i: (0, 0))
    vec = pl.BlockSpec((1, MD), lambda i: (0, 0))
    return pl.pallas_call(
        body,
        grid=(NN // _NF_BLK,),
        in_specs=[blk, blk, blk, full, full, vec],
        out_specs=blk,
        out_shape=jax.ShapeDtypeStruct((NN, MD), jnp.float32),
    )(inc0, inc1, atom, WaT, WcT, b)


_ED_BLK = 2000
_HBLKS = NEH // _ED_BLK


def _edge_epilogue_half(h, msg, gathered_h, bond, WbT, gamma, beta, prev):
    def body(m_ref, g_ref, bo_ref, wb_ref, ga_ref, be_ref, *rest):
        o_ref = rest[-1]
        bp = jnp.dot(bo_ref[...], wb_ref[...], preferred_element_type=jnp.float32)
        upd = jnp.maximum(g_ref[...] + bp, 0.0)
        x = m_ref[...] + upd
        mu = jnp.mean(x, axis=1, keepdims=True)
        xc = x - mu
        var = jnp.mean(xc * xc, axis=1, keepdims=True)
        o_ref[...] = xc * lax.rsqrt(var + 1e-5) * ga_ref[...] + be_ref[...]

    blk = pl.BlockSpec((_ED_BLK, MD), lambda i: (h * _HBLKS + i, 0))
    gblk = pl.BlockSpec((_ED_BLK, MD), lambda i: (i, 0))
    bblk = pl.BlockSpec((_ED_BLK, BD), lambda i: (h * _HBLKS + i, 0))
    wblk = pl.BlockSpec((BD, MD), lambda i: (0, 0))
    vec = pl.BlockSpec((1, MD), lambda i: (0, 0))
    in_specs = [blk, gblk, bblk, wblk, vec, vec]
    args = [msg, gathered_h, bond, WbT, gamma, beta]
    aliases = {}
    if prev is not None:
        in_specs.append(pl.BlockSpec(memory_space=pl.ANY))
        args.append(prev)
        aliases = {6: 0}
    return pl.pallas_call(
        body,
        grid=(_HBLKS,),
        in_specs=in_specs,
        out_specs=blk,
        out_shape=jax.ShapeDtypeStruct((NE, MD), jnp.float32),
        input_output_aliases=aliases,
    )(*args)


def kernel(msg, atom, bond, src, dst, W, b, gamma, beta):
    src3 = src.astype(jnp.int32).reshape(2, NW, EPH)
    dst3 = dst.astype(jnp.int32).reshape(NW, NITER, CHUNK)
    WaT = W[:, :AD].T
    WbT = W[:, AD:AD + BD].T
    WcT = W[:, AD + BD:].T
    zeros = jnp.zeros((NN, MD), jnp.float32)
    gamma2 = gamma.reshape(1, MD)
    beta2 = beta.reshape(1, MD)

    inc2 = _scatter_add(msg, dst3, zeros)
    nf = _node_features(inc2[:NN], inc2[NN:], atom, WaT, WcT,
                        b.reshape(1, MD))
    g0 = _gather_half(nf, src3[0])
    g1 = _gather_half(nf, src3[1])
    out = _edge_epilogue_half(0, msg, g0, bond, WbT, gamma2, beta2, None)
    out = _edge_epilogue_half(1, msg, g1, bond, WbT, gamma2, beta2, out)
    return out


# speedup vs baseline: 1.2753x; 1.1204x over previous
import functools

import jax
import jax.numpy as jnp
from jax import lax
from jax.experimental import pallas as pl
from jax.experimental.pallas import tpu as pltpu
from jax.experimental.pallas import tpu_sc as plsc

NN = 10000
NE = 320000
MD = 128
AD = 128
BD = 16

NC = 2
NS = 16
NW = NC * NS
EPW = NE // NW
CHUNK = 80
NITER = EPW // CHUNK


def _sc_mesh():
    return plsc.VectorSubcoreMesh(core_axis_name="c", subcore_axis_name="s")


_NBUF = 3


def _scatter_add(msg, dst3, zeros):
    @functools.partial(
        pl.kernel,
        mesh=_sc_mesh(),
        out_type=jax.ShapeDtypeStruct((NC * NN, MD), jnp.float32),
        scratch_types=[
            pltpu.VMEM((NITER, CHUNK), jnp.int32),
            *[pltpu.VMEM((CHUNK, MD), jnp.float32) for _ in range(_NBUF)],
            *[pltpu.SemaphoreType.DMA for _ in range(_NBUF)],
            pltpu.VMEM_SHARED((NN, MD), jnp.float32),
        ],
    )
    def k(msg_hbm, dst_hbm, zeros_hbm, out_hbm, idx_v, *rest):
        bufs = rest[:_NBUF]
        sems = rest[_NBUF:2 * _NBUF]
        inc_sh = rest[2 * _NBUF]
        cid = lax.axis_index("c")
        sid = lax.axis_index("s")
        wid = sid * NC + cid
        rpt = 1000

        @pl.when(sid < NN // rpt)
        def _():
            pltpu.sync_copy(zeros_hbm.at[pl.ds(sid * rpt, rpt)],
                            inc_sh.at[pl.ds(sid * rpt, rpt)])

        pltpu.sync_copy(dst_hbm.at[wid], idx_v)
        plsc.subcore_barrier()

        loads = [None] * _NBUF

        def start_load(b, i):
            base = wid * EPW + i * CHUNK
            loads[b] = pltpu.async_copy(
                msg_hbm.at[pl.ds(base, CHUNK)], bufs[b], sems[b])

        for b in range(_NBUF):
            start_load(b, b)
        for i in range(NITER):
            b = i % _NBUF
            loads[b].wait()
            pltpu.sync_copy(bufs[b], inc_sh.at[idx_v.at[i]], add=True)
            if i + _NBUF < NITER:
                start_load(b, i + _NBUF)

        plsc.subcore_barrier()

        @pl.when(sid < NN // rpt)
        def _():
            pltpu.sync_copy(inc_sh.at[pl.ds(sid * rpt, rpt)],
                            out_hbm.at[pl.ds(cid * NN + sid * rpt, rpt)])

    return k(msg, dst3, zeros)


NEH = NE // 2
NRH = NEH // 2
_RPW_BIG = 2504
_RPW_SMALL = 2496
_GROWS = 400


def _gather_half_packed(nfp, srcA, srcB):
    @functools.partial(
        pl.kernel,
        mesh=_sc_mesh(),
        compiler_params=pltpu.CompilerParams(use_tc_tiling_on_sc=False),
        out_type=jax.ShapeDtypeStruct((NRH, MD), jnp.int32),
        scratch_types=[
            pltpu.VMEM((_RPW_BIG,), jnp.int32),
            pltpu.VMEM((_RPW_BIG,), jnp.int32),
            pltpu.VMEM_SHARED((NS, _GROWS, MD), jnp.int32),
            *[pltpu.VMEM((80, _HD), jnp.int32) for _ in range(4)],
            pltpu.SemaphoreType.DMA,
            pltpu.SemaphoreType.DMA,
            pltpu.SemaphoreType.DMA,
        ],
    )
    def k(nf_hbm, srcA_hbm, srcB_hbm, out_hbm,
          idxA_v, idxB_v, obuf_sh, gA0, gB0, gA1, gB1,
          gsem, csem, wsem):
        cid = lax.axis_index("c")
        sid = lax.axis_index("s")
        wid = sid * NC + cid
        gbufs = ((gA0, gB0), (gA1, gB1))

        def run(row_start, tail_sz):
            nrows = 31 * 80 + tail_sz
            pltpu.sync_copy(srcA_hbm.at[pl.ds(row_start, nrows)],
                            idxA_v.at[pl.ds(0, nrows)])
            pltpu.sync_copy(srcB_hbm.at[pl.ds(row_start, nrows)],
                            idxB_v.at[pl.ds(0, nrows)])
            sizes = [80] * 31 + [tail_sz]
            wh = None
            merges = [None, None]
            for gi in range(7):
                if wh is not None:
                    wh.wait()
                chunks = list(range(5 * gi, 5 * gi + 5)) if gi < 6 else [30, 31]
                boff = 0
                for ci, c in enumerate(chunks):
                    csz = sizes[c]
                    p = ci & 1
                    if merges[p] is not None:
                        for m in merges[p]:
                            m.wait()
                        merges[p] = None
                    ga, gb = gbufs[p]
                    ha = pltpu.async_copy(
                        nf_hbm.at[idxA_v.at[pl.ds(c * 80, csz)]],
                        ga.at[pl.ds(0, csz)], gsem)
                    hb = pltpu.async_copy(
                        nf_hbm.at[idxB_v.at[pl.ds(c * 80, csz)]],
                        gb.at[pl.ds(0, csz)], gsem)
                    ha.wait()
                    hb.wait()
                    merges[p] = [
                        pltpu.async_copy(
                            ga.at[pl.ds(0, csz)],
                            obuf_sh.at[sid, pl.ds(boff, csz), pl.ds(0, _HD)],
                            csem),
                        pltpu.async_copy(
                            gb.at[pl.ds(0, csz)],
                            obuf_sh.at[sid, pl.ds(boff, csz), pl.ds(_HD, _HD)],
                            csem),
                    ]
                    boff += csz
                for p in range(2):
                    if merges[p] is not None:
                        for m in merges[p]:
                            m.wait()
                        merges[p] = None
                wh = pltpu.async_copy(
                    obuf_sh.at[sid, pl.ds(0, boff)],
                    out_hbm.at[pl.ds(row_start + 400 * gi, boff)], wsem)
            if wh is not None:
                wh.wait()

        @pl.when(wid < 16)
        def _():
            run(wid * _RPW_BIG, 24)

        @pl.when(wid >= 16)
        def _():
            run(16 * _RPW_BIG + (wid - 16) * _RPW_SMALL, 16)

    return k(nfp, srcA, srcB)


_NF_BLK = 2000


_HD = MD // 2


def _rne_bf16_bits(f):
    u = lax.bitcast_convert_type(f, jnp.uint32)
    return ((u + jnp.uint32(0x7FFF) + ((u >> 16) & jnp.uint32(1))) >> 16
            ).astype(jnp.int32)


def _node_features(inc0, inc1, atom, WaT, WcT, b):
    def body(i0_ref, i1_ref, a_ref, wa_ref, wc_ref, b_ref, o_ref):
        inc = i0_ref[...] + i1_ref[...]
        nf = (
            jnp.dot(a_ref[...], wa_ref[...], preferred_element_type=jnp.float32)
            + jnp.dot(inc, wc_ref[...], preferred_element_type=jnp.float32)
            + b_ref[...]
        )
        lo = _rne_bf16_bits(nf[:, :_HD])
        hi = _rne_bf16_bits(nf[:, _HD:])
        o_ref[...] = (hi << 16) | lo

    blk = pl.BlockSpec((_NF_BLK, MD), lambda i: (i, 0))
    full = pl.BlockSpec((MD, MD), lambda i: (0, 0))
    vec = pl.BlockSpec((1, MD), lambda i: (0, 0))
    return pl.pallas_call(
        body,
        grid=(NN // _NF_BLK,),
        in_specs=[blk, blk, blk, full, full, vec],
        out_specs=pl.BlockSpec((_NF_BLK, _HD), lambda i: (i, 0)),
        out_shape=jax.ShapeDtypeStruct((NN, _HD), jnp.int32),
    )(inc0, inc1, atom, WaT, WcT, b)


_ED_BLK = 2000
_HBLKS = NEH // _ED_BLK


def _edge_epilogue_half(h, msg, gathered_h, bond, WbT, gamma, beta, prev):
    def body(m_ref, g_ref, bo_ref, wb_ref, ga_ref, be_ref, *rest):
        o_ref = rest[-1]
        bp = jnp.dot(bo_ref[...], wb_ref[...], preferred_element_type=jnp.float32)
        w = g_ref[...]
        f_lo = lax.bitcast_convert_type(w << 16, jnp.float32)
        f_hi = lax.bitcast_convert_type(w & jnp.int32(-65536), jnp.float32)
        a_rows = jnp.concatenate([f_lo[:, :_HD], f_hi[:, :_HD]], axis=1)
        b_rows = jnp.concatenate([f_lo[:, _HD:], f_hi[:, _HD:]], axis=1)
        nf_rows = jnp.concatenate([a_rows, b_rows], axis=0)
        upd = jnp.maximum(nf_rows + bp, 0.0)
        x = m_ref[...] + upd
        mu = jnp.mean(x, axis=1, keepdims=True)
        xc = x - mu
        var = jnp.mean(xc * xc, axis=1, keepdims=True)
        o_ref[...] = xc * lax.rsqrt(var + 1e-5) * ga_ref[...] + be_ref[...]

    blk = pl.BlockSpec((_ED_BLK, MD), lambda i: (h * _HBLKS + i, 0))
    gblk = pl.BlockSpec((_ED_BLK // 2, MD), lambda i: (i, 0))
    bblk = pl.BlockSpec((_ED_BLK, BD), lambda i: (h * _HBLKS + i, 0))
    wblk = pl.BlockSpec((BD, MD), lambda i: (0, 0))
    vec = pl.BlockSpec((1, MD), lambda i: (0, 0))
    in_specs = [blk, gblk, bblk, wblk, vec, vec]
    args = [msg, gathered_h, bond, WbT, gamma, beta]
    aliases = {}
    if prev is not None:
        in_specs.append(pl.BlockSpec(memory_space=pl.ANY))
        args.append(prev)
        aliases = {6: 0}
    return pl.pallas_call(
        body,
        grid=(_HBLKS,),
        in_specs=in_specs,
        out_specs=blk,
        out_shape=jax.ShapeDtypeStruct((NE, MD), jnp.float32),
        input_output_aliases=aliases,
    )(*args)


def kernel(msg, atom, bond, src, dst, W, b, gamma, beta):
    srcp = src.astype(jnp.int32).reshape(2, NEH // 2000, 2, 1000)
    srcA = [srcp[h, :, 0, :].reshape(NRH) for h in range(2)]
    srcB = [srcp[h, :, 1, :].reshape(NRH) for h in range(2)]
    dst3 = dst.astype(jnp.int32).reshape(NW, NITER, CHUNK)
    WaT = W[:, :AD].T
    WbT = W[:, AD:AD + BD].T
    WcT = W[:, AD + BD:].T
    zeros = jnp.zeros((NN, MD), jnp.float32)
    gamma2 = gamma.reshape(1, MD)
    beta2 = beta.reshape(1, MD)

    inc2 = _scatter_add(msg, dst3, zeros)
    nfp = _node_features(inc2[:NN], inc2[NN:], atom, WaT, WcT,
                         b.reshape(1, MD))
    g0 = _gather_half_packed(nfp, srcA[0], srcB[0])
    g1 = _gather_half_packed(nfp, srcA[1], srcB[1])
    out = _edge_epilogue_half(0, msg, g0, bond, WbT, gamma2, beta2, None)
    out = _edge_epilogue_half(1, msg, g1, bond, WbT, gamma2, beta2, out)
    return out
